# baseline (speedup 1.0000x reference)
import functools

import jax
import jax.numpy as jnp
from jax import lax
from jax.experimental import pallas as pl
from jax.experimental.pallas import tpu as pltpu
from jax.experimental.pallas import tpu_sc as plsc

VOCAB_SIZE = 1000000
TAG_VOCAB = 1000
REL_VOCAB = 1000
WORD_DIM = 64
TAG_DIM = 32
REL_DIM = 32
B = 4096
L = 200
N = B * L

NC = 2
NS = 16
NW = NC * NS
BB = B // 128
LO = L // 8
NU_TOT = L * BB
NU = NU_TOT // NW

_mesh = plsc.VectorSubcoreMesh(
    core_axis_name="c", subcore_axis_name="s", num_cores=NC, num_subcores=NS
)


@functools.partial(
    pl.kernel,
    out_type=(
        jax.ShapeDtypeStruct((L, WORD_DIM // 8, BB, 8, 128), jnp.float32),
        jax.ShapeDtypeStruct((L, TAG_DIM // 8, BB, 8, 128), jnp.float32),
        jax.ShapeDtypeStruct((L, REL_DIM // 8, BB, 8, 128), jnp.float32),
    ),
    mesh=_mesh,
    scratch_types=(
        pltpu.VMEM((2, 1, 128), jnp.int32),
        pltpu.VMEM((2, 1, 128), jnp.int32),
        pltpu.VMEM((2, 1, 128), jnp.int32),
        pltpu.VMEM((2, 128, WORD_DIM), jnp.float32),
        pltpu.VMEM((2, WORD_DIM, 128), jnp.float32),
        pltpu.VMEM((2, TAG_DIM, 128), jnp.float32),
        pltpu.VMEM((2, REL_DIM, 128), jnp.float32),
        pltpu.VMEM((TAG_VOCAB, TAG_DIM), jnp.float32),
        pltpu.VMEM((REL_VOCAB, REL_DIM), jnp.float32),
        pltpu.SemaphoreType.DMA,
        pltpu.SemaphoreType.DMA,
        pltpu.SemaphoreType.DMA,
        pltpu.SemaphoreType.DMA,
        pltpu.SemaphoreType.DMA,
        pltpu.SemaphoreType.DMA,
    ),
    compiler_params=pltpu.CompilerParams(
        needs_layout_passes=False, use_tc_tiling_on_sc=False
    ),
)
def _emb3(sent_hbm, tag_hbm, rel_hbm, wtab_hbm, ttab_hbm, rtab_hbm,
          wout_hbm, tout_hbm, rout_hbm,
          widx, tidx, ridx, gw, tw, tt, tr, ttab_v, rtab_v,
          si0, si1, sg0, sg1, so0, so1):
    wid = lax.axis_index("s") * NC + lax.axis_index("c")
    r0 = wid * NU
    si = (si0, si1)
    sg = (sg0, sg1)
    so = (so0, so1)

    pltpu.sync_copy(ttab_hbm, ttab_v)
    pltpu.sync_copy(rtab_hbm, rtab_v)

    def fire_idx(b, c):
        r = r0 + c
        pltpu.async_copy(sent_hbm.at[pl.ds(r, 1)], widx.at[b], si[b])
        pltpu.async_copy(tag_hbm.at[pl.ds(r, 1)], tidx.at[b], si[b])
        pltpu.async_copy(rel_hbm.at[pl.ds(r, 1)], ridx.at[b], si[b])

    def wait_idx(b):
        pltpu.make_async_copy(sent_hbm.at[pl.ds(0, 1)], widx.at[b], si[b]).wait()
        pltpu.make_async_copy(tag_hbm.at[pl.ds(0, 1)], tidx.at[b], si[b]).wait()
        pltpu.make_async_copy(rel_hbm.at[pl.ds(0, 1)], ridx.at[b], si[b]).wait()

    def fire_g(b):
        pltpu.async_copy(wtab_hbm.at[widx.at[b].at[0]], gw.at[b], sg[b])

    def wait_g(b):
        pltpu.make_async_copy(wtab_hbm.at[widx.at[b].at[0]], gw.at[b], sg[b]).wait()

    def unit_lbt(c):
        r = r0 + c
        lo = r // 256
        bt = (r // 8) % 32
        li = r % 8
        return lo * 8 + li, bt

    def fire_writes(b, c):
        l, bt = unit_lbt(c)
        for dt in range(WORD_DIM // 8):
            pltpu.async_copy(
                tw.at[b].at[pl.ds(dt * 8, 8)],
                wout_hbm.at[l].at[dt].at[bt], so[b],
            )
        for dt in range(TAG_DIM // 8):
            pltpu.async_copy(
                tt.at[b].at[pl.ds(dt * 8, 8)],
                tout_hbm.at[l].at[dt].at[bt], so[b],
            )
        for dt in range(REL_DIM // 8):
            pltpu.async_copy(
                tr.at[b].at[pl.ds(dt * 8, 8)],
                rout_hbm.at[l].at[dt].at[bt], so[b],
            )

    def wait_writes(b):
        for dt in range(WORD_DIM // 8):
            pltpu.make_async_copy(
                tw.at[b].at[pl.ds(dt * 8, 8)],
                wout_hbm.at[0].at[dt].at[0], so[b],
            ).wait()
        for dt in range(TAG_DIM // 8):
            pltpu.make_async_copy(
                tt.at[b].at[pl.ds(dt * 8, 8)],
                tout_hbm.at[0].at[dt].at[0], so[b],
            ).wait()
        for dt in range(REL_DIM // 8):
            pltpu.make_async_copy(
                tr.at[b].at[pl.ds(dt * 8, 8)],
                rout_hbm.at[0].at[dt].at[0], so[b],
            ).wait()

    def compute(b):
        def bg_body(bg, _):
            rowi = bg * 16 + lax.iota(jnp.int32, 16)
            ivw = widx[b, 0, pl.ds(bg * 16, 16)]
            maskw = ivw == 0
            nz = plsc.all_reduce_population_count(maskw)

            @pl.when(nz[0] > 0)
            def _():
                zz = jnp.zeros((16,), jnp.float32)
                for col in range(WORD_DIM):
                    plsc.store_scatter(
                        gw.at[b],
                        [rowi, jnp.full((16,), col, jnp.int32)], zz,
                        mask=maskw,
                    )
            for d in range(WORD_DIM):
                v = plsc.load_gather(
                    gw.at[b], [rowi, jnp.full((16,), d, jnp.int32)]
                )
                tw[b, d, pl.ds(bg * 16, 16)] = v
            ivt = tidx[b, 0, pl.ds(bg * 16, 16)]
            for d in range(TAG_DIM):
                v = plsc.load_gather(
                    ttab_v, [ivt, jnp.full((16,), d, jnp.int32)]
                )
                tt[b, d, pl.ds(bg * 16, 16)] = v
            ivr = ridx[b, 0, pl.ds(bg * 16, 16)]
            for d in range(REL_DIM):
                v = plsc.load_gather(
                    rtab_v, [ivr, jnp.full((16,), d, jnp.int32)]
                )
                tr[b, d, pl.ds(bg * 16, 16)] = v
            return 0

        lax.fori_loop(0, 8, bg_body, 0)

    pltpu.sync_copy(sent_hbm.at[pl.ds(r0, 1)], widx.at[0])
    pltpu.sync_copy(tag_hbm.at[pl.ds(r0, 1)], tidx.at[0])
    pltpu.sync_copy(rel_hbm.at[pl.ds(r0, 1)], ridx.at[0])
    fire_g(0)
    fire_idx(1, 1)

    def body(k, _):
        c0 = 2 * k
        c1 = c0 + 1
        wait_g(0)
        wait_idx(1)
        fire_g(1)

        @pl.when(k > 0)
        def _():
            wait_writes(0)

        compute(0)
        fire_writes(0, c0)

        @pl.when(c0 + 2 < NU)
        def _():
            fire_idx(0, c0 + 2)

        wait_g(1)

        @pl.when(c1 + 1 < NU)
        def _():
            wait_idx(0)
            fire_g(0)

        @pl.when(k > 0)
        def _():
            wait_writes(1)

        compute(1)
        fire_writes(1, c1)

        @pl.when(c1 + 2 < NU)
        def _():
            fire_idx(1, c1 + 2)

        return 0

    lax.fori_loop(0, NU // 2, body, 0)
    wait_writes(0)
    wait_writes(1)


def _phys_view(x):
    return (
        x.reshape(BB, 128, LO, 8).transpose(2, 0, 3, 1).reshape(NU_TOT, 128)
        .astype(jnp.int32)
    )


def _logical_out(a, d):
    return a.transpose(2, 4, 0, 1, 3).reshape(B, L, d)[:, None]


def kernel(sent_inputs, tag_inputs, rel_inputs, word_table, tag_table, rel_table):
    wout, tout, rout = _emb3(
        _phys_view(sent_inputs), _phys_view(tag_inputs), _phys_view(rel_inputs),
        word_table, tag_table, rel_table,
    )
    return (
        _logical_out(wout, WORD_DIM),
        _logical_out(tout, TAG_DIM),
        _logical_out(rout, REL_DIM),
    )
